# Initial kernel scaffold; baseline (speedup 1.0000x reference)
import jax
import jax.numpy as jnp
from jax.experimental import pallas as pl


def kernel(x, norm1_scale, qkv_W, qkv_b, out_W, out_b, norm2_scale, gate_W, gate_b, lin1_W, lin1_b, lin2_W, lin2_b, proj_W, proj_b):
    raise NotImplementedError("write your pallas kernel here")


# speedup vs baseline: 1.3924x; 1.3924x over previous
import math
import functools

import jax
import jax.numpy as jnp
from jax.experimental import pallas as pl
from jax.experimental.pallas import tpu as pltpu

T, D = 2048, 768
H, HD = 12, 64
E, K, FF = 8, 2, 3072
HALF = HD // 2

_EPS = 1e-5


def _rms(x, scale, eps=_EPS):
    norm = jnp.sqrt(jnp.mean(x * x, axis=-1, keepdims=True))
    return x / (norm + eps) * scale


def _qkv_body(x_ref, scale_ref, w_ref, b_ref, o_ref):
    h = _rms(x_ref[...], scale_ref[...])
    o_ref[...] = jnp.dot(h, w_ref[...], preferred_element_type=jnp.float32) + b_ref[...]


def _qkv(x, scale, w, b):
    BT = 512
    return pl.pallas_call(
        _qkv_body,
        grid=(T // BT,),
        in_specs=[
            pl.BlockSpec((BT, D), lambda i: (i, 0)),
            pl.BlockSpec((1, D), lambda i: (0, 0)),
            pl.BlockSpec((D, 3 * D), lambda i: (0, 0)),
            pl.BlockSpec((1, 3 * D), lambda i: (0, 0)),
        ],
        out_specs=pl.BlockSpec((BT, 3 * D), lambda i: (i, 0)),
        out_shape=jax.ShapeDtypeStruct((T, 3 * D), jnp.float32),
    )(x, scale, w, b)


def _rope_apply(x, sin, cos):
    x1 = x[:, :HALF]
    x2 = x[:, HALF:]
    return jnp.concatenate([x1 * cos - x2 * sin, x1 * sin + x2 * cos], axis=-1)


def _attn_body(q_ref, k_ref, v_ref, sq_ref, cq_ref, sk_ref, ck_ref, o_ref, *, qb):
    qi = pl.program_id(1)
    for sub in range(2):
        sl = slice(sub * HD, (sub + 1) * HD)
        q = _rope_apply(q_ref[:, sl], sq_ref[...], cq_ref[...])
        k = _rope_apply(k_ref[:, sl], sk_ref[...], ck_ref[...])
        s = jax.lax.dot_general(q, k, (((1,), (1,)), ((), ())),
                                preferred_element_type=jnp.float32)
        s = s * (1.0 / math.sqrt(HD))
        rows = qi * qb + jax.lax.broadcasted_iota(jnp.int32, s.shape, 0)
        cols = jax.lax.broadcasted_iota(jnp.int32, s.shape, 1)
        s = jnp.where(cols > rows, -1e30, s)
        m = jnp.max(s, axis=-1, keepdims=True)
        p = jnp.exp(s - m)
        p = p / jnp.sum(p, axis=-1, keepdims=True)
        o = jax.lax.dot_general(p, v_ref[:, sl], (((1,), (0,)), ((), ())),
                                preferred_element_type=jnp.float32)
        o_ref[:, sl] = o * (1.0 / math.sqrt(H))


def _attention(qkv, sin, cos):
    QB = 512
    HP = H // 2
    return pl.pallas_call(
        functools.partial(_attn_body, qb=QB),
        grid=(HP, T // QB),
        in_specs=[
            pl.BlockSpec((QB, 2 * HD), lambda h, qi: (qi, h)),
            pl.BlockSpec((T, 2 * HD), lambda h, qi: (0, HP + h)),
            pl.BlockSpec((T, 2 * HD), lambda h, qi: (0, 2 * HP + h)),
            pl.BlockSpec((QB, HALF), lambda h, qi: (qi, 0)),
            pl.BlockSpec((QB, HALF), lambda h, qi: (qi, 0)),
            pl.BlockSpec((T, HALF), lambda h, qi: (0, 0)),
            pl.BlockSpec((T, HALF), lambda h, qi: (0, 0)),
        ],
        out_specs=pl.BlockSpec((QB, 2 * HD), lambda h, qi: (qi, h)),
        out_shape=jax.ShapeDtypeStruct((T, D), jnp.float32),
    )(qkv, qkv, qkv, sin, cos, sin, cos)


def _proj_body(a_ref, w_ref, b_ref, x_ref, o_ref):
    o_ref[...] = (jnp.dot(a_ref[...], w_ref[...], preferred_element_type=jnp.float32)
                  + b_ref[...] + x_ref[...])


def _out_proj(attn, w, b, x):
    BT = 512
    return pl.pallas_call(
        _proj_body,
        grid=(T // BT,),
        in_specs=[
            pl.BlockSpec((BT, D), lambda i: (i, 0)),
            pl.BlockSpec((D, D), lambda i: (0, 0)),
            pl.BlockSpec((1, D), lambda i: (0, 0)),
            pl.BlockSpec((BT, D), lambda i: (i, 0)),
        ],
        out_specs=pl.BlockSpec((BT, D), lambda i: (i, 0)),
        out_shape=jax.ShapeDtypeStruct((T, D), jnp.float32),
    )(attn, w, b, x)


def _gate_body(y_ref, scale_ref, gw_ref, gb_ref, h2_ref, coeff_ref, aux_ref):
    h2 = _rms(y_ref[...], scale_ref[...])
    h2_ref[...] = h2
    logits = jnp.dot(h2, gw_ref[...], preferred_element_type=jnp.float32) + gb_ref[...]
    eidx = jax.lax.broadcasted_iota(jnp.int32, logits.shape, 1)
    m0 = jnp.max(logits, axis=-1, keepdims=True)
    i0 = jnp.min(jnp.where(logits == m0, eidx, E), axis=-1, keepdims=True)
    l2 = jnp.where(eidx == i0, -jnp.inf, logits)
    m1 = jnp.max(l2, axis=-1, keepdims=True)
    i1 = jnp.min(jnp.where(l2 == m1, eidx, E), axis=-1, keepdims=True)
    z = jnp.exp(m1 - m0)
    p0 = 1.0 / (1.0 + z)
    p1 = 1.0 - p0
    sel0 = (eidx == i0).astype(jnp.float32)
    sel1 = (eidx == i1).astype(jnp.float32)
    coeff_ref[...] = sel0 * p0 + sel1 * p1
    counts = jnp.sum(sel0 + sel1, axis=0, keepdims=True)
    frac = counts / jnp.sum(counts)
    aux_ref[...] = jnp.sum((frac - 1.0 / E) ** 2, keepdims=True)


def _gate(y, scale, gw, gb):
    return pl.pallas_call(
        _gate_body,
        grid=(1,),
        in_specs=[
            pl.BlockSpec((T, D), lambda i: (0, 0)),
            pl.BlockSpec((1, D), lambda i: (0, 0)),
            pl.BlockSpec((D, E), lambda i: (0, 0)),
            pl.BlockSpec((1, E), lambda i: (0, 0)),
        ],
        out_specs=[
            pl.BlockSpec((T, D), lambda i: (0, 0)),
            pl.BlockSpec((T, E), lambda i: (0, 0)),
            pl.BlockSpec((1, 1), lambda i: (0, 0)),
        ],
        out_shape=[
            jax.ShapeDtypeStruct((T, D), jnp.float32),
            jax.ShapeDtypeStruct((T, E), jnp.float32),
            jax.ShapeDtypeStruct((1, 1), jnp.float32),
        ],
    )(y, scale, gw, gb)


def _moe_body(w1_ref, w2_ref, b1_ref, b2_ref, w3_ref, b3_ref,
              h2_ref, coeff_ref, y_ref, o_ref, acc_ref, *, nf, bt):
    e = pl.program_id(0)
    f = pl.program_id(1)
    i = pl.program_id(2)
    ni = pl.num_programs(2)
    xb = h2_ref[...]
    h1 = jnp.dot(xb, w1_ref[0], preferred_element_type=jnp.float32) + b1_ref[0]
    h1 = h1 * jax.lax.logistic(h1)
    h2m = jnp.dot(xb, w2_ref[0], preferred_element_type=jnp.float32) + b2_ref[0]
    he = h1 * h2m
    part = jnp.dot(he, w3_ref[0], preferred_element_type=jnp.float32)
    cb = coeff_ref[...]
    lane = jax.lax.broadcasted_iota(jnp.int32, cb.shape, 1)
    c = jnp.sum(jnp.where(lane == e, cb, 0.0), axis=-1, keepdims=True)
    upd = c * part

    @pl.when(jnp.logical_and(e == 0, f == 0))
    def _():
        acc_ref[pl.ds(i * bt, bt), :] = upd

    @pl.when(jnp.logical_not(jnp.logical_and(e == 0, f == 0)))
    def _():
        acc_ref[pl.ds(i * bt, bt), :] += upd

    @pl.when(f == nf - 1)
    def _():
        acc_ref[pl.ds(i * bt, bt), :] += c * b3_ref[0]

    @pl.when(jnp.logical_and(e == E - 1, jnp.logical_and(f == nf - 1, i == ni - 1)))
    def _():
        o_ref[...] = acc_ref[...] + y_ref[...]


def _moe(h2, coeff, y, w1, b1, w2, b2, w3, b3):
    FFT = 1024
    BT = 512
    nf = FF // FFT
    body = functools.partial(_moe_body, nf=nf, bt=BT)
    return pl.pallas_call(
        body,
        grid=(E, nf, T // BT),
        in_specs=[
            pl.BlockSpec((1, D, FFT), lambda e, f, i: (e, 0, f)),
            pl.BlockSpec((1, D, FFT), lambda e, f, i: (e, 0, f)),
            pl.BlockSpec((1, 1, FFT), lambda e, f, i: (e, 0, f)),
            pl.BlockSpec((1, 1, FFT), lambda e, f, i: (e, 0, f)),
            pl.BlockSpec((1, FFT, D), lambda e, f, i: (e, f, 0)),
            pl.BlockSpec((1, 1, D), lambda e, f, i: (e, 0, 0)),
            pl.BlockSpec((BT, D), lambda e, f, i: (i, 0)),
            pl.BlockSpec((BT, E), lambda e, f, i: (i, 0)),
            pl.BlockSpec((T, D), lambda e, f, i: (0, 0)),
        ],
        out_specs=pl.BlockSpec((T, D), lambda e, f, i: (0, 0)),
        out_shape=jax.ShapeDtypeStruct((T, D), jnp.float32),
        scratch_shapes=[pltpu.VMEM((T, D), jnp.float32)],
    )(w1, w2, b1.reshape(E, 1, FF), b2.reshape(E, 1, FF), w3,
      b3.reshape(E, 1, D), h2, coeff, y)


def kernel(x, norm1_scale, qkv_W, qkv_b, out_W, out_b, norm2_scale,
           gate_W, gate_b, lin1_W, lin1_b, lin2_W, lin2_b, proj_W, proj_b):
    b = x.shape[0]
    x2 = x.reshape(T, D)

    freq = jnp.arange(HALF, dtype=jnp.float32)
    theta = 1.0 / (10000.0 ** (2.0 * freq / HD))
    pos = jnp.arange(T, dtype=jnp.float32)
    angles = pos[:, None] * theta[None, :]
    sin = jnp.sin(angles)
    cos = jnp.cos(angles)

    qkv = _qkv(x2, norm1_scale.reshape(1, D), qkv_W, qkv_b.reshape(1, 3 * D))
    attn = _attention(qkv, sin, cos)
    y = _out_proj(attn, out_W, out_b.reshape(1, D), x2)
    h2, coeff, aux = _gate(y, norm2_scale.reshape(1, D), gate_W,
                           gate_b.reshape(1, E))
    x_out = _moe(h2, coeff, y, lin1_W, lin1_b, lin2_W, lin2_b, proj_W, proj_b)
    return (x_out.reshape(b, T, D), aux.reshape(()))
